# baseline (speedup 1.0000x reference)
import dataclasses
import functools

import jax
import jax.numpy as jnp
from jax import lax
from jax.experimental import pallas as pl
from jax.experimental.pallas import tpu as pltpu
from jax.experimental.pallas import tpu_sc as plsc

_L = 16
_CH = 128
_D = 128


def _norm_body(x_ref, o_ref):
    x = x_ref[...]
    ss = jnp.sum(x * x, axis=1, keepdims=True)
    n = jnp.sqrt(ss)
    o_ref[...] = x / jnp.maximum(n, 1e-12)


def _normalize_rows(x):
    return pl.pallas_call(
        _norm_body,
        out_shape=jax.ShapeDtypeStruct(x.shape, x.dtype),
    )(x)


def _bce_body(n_valid, p_ref, t_ref, o_ref):
    p = p_ref[...]
    t = t_ref[...]
    term = jnp.maximum(p, 0.0) - p * t + jnp.log1p(jnp.exp(-jnp.abs(p)))
    rows, cols = p.shape
    idx = (lax.broadcasted_iota(jnp.int32, (rows, cols), 0) * cols
           + lax.broadcasted_iota(jnp.int32, (rows, cols), 1))
    term = jnp.where(idx < n_valid, term, 0.0)
    o_ref[...] = jnp.sum(term, axis=0, keepdims=True)


def _bce_sum(p2d, t2d, n_valid):
    part = pl.pallas_call(
        functools.partial(_bce_body, n_valid),
        out_shape=jax.ShapeDtypeStruct((1, p2d.shape[1]), jnp.float32),
    )(p2d, t2d)
    return jnp.sum(part)


def _make_edge_kernel(nw, ppi_chunks, dti_chunks):
    ppw = ppi_chunks * _CH
    dtw = dti_chunks * _CH
    mesh = plsc.VectorSubcoreMesh(core_axis_name="c", subcore_axis_name="s")
    info = plsc.get_sparse_core_info()
    nc = info.num_cores

    cp = pltpu.CompilerParams()
    if "needs_layout_passes" in pltpu.CompilerParams.__dataclass_fields__:
        cp = dataclasses.replace(cp, needs_layout_passes=False)

    @functools.partial(
        pl.kernel,
        mesh=mesh,
        compiler_params=cp,
        out_type=jax.ShapeDtypeStruct((nw, 2, _L), jnp.float32),
        scratch_types=[
            pltpu.VMEM((_CH,), jnp.int32),
            pltpu.VMEM((_CH,), jnp.int32),
            pltpu.VMEM((_CH, _D), jnp.float32),
            pltpu.VMEM((_CH, _D), jnp.float32),
            pltpu.VMEM((2, _L), jnp.float32),
            pltpu.SemaphoreType.DMA,
            pltpu.SemaphoreType.DMA,
        ],
    )
    def edge_kernel(gene_hbm, drug_hbm, ps_hbm, pd_hbm, ds_hbm, dd_hbm,
                    out_hbm, sidx, didx, srows, drows, ovec, sem_a, sem_b):
        wid = lax.axis_index("s") * nc + lax.axis_index("c")

        def chunk_sum(src_tbl, dst_tbl, sidx_hbm, didx_hbm, base, acc):
            pltpu.sync_copy(sidx_hbm.at[pl.ds(base, _CH)], sidx)
            pltpu.sync_copy(didx_hbm.at[pl.ds(base, _CH)], didx)
            ca = pltpu.async_copy(src_tbl.at[sidx], srows, sem_a)
            cb = pltpu.async_copy(dst_tbl.at[didx], drows, sem_b)
            ca.wait()
            cb.wait()

            def edge(e, acc):
                prod = srows[e, pl.ds(0, _L)] * drows[e, pl.ds(0, _L)]
                for k in range(1, _D // _L):
                    prod = prod + (srows[e, pl.ds(k * _L, _L)]
                                   * drows[e, pl.ds(k * _L, _L)])
                dt = jnp.sum(prod)
                r = dt - 1.0
                return acc + r * r

            return lax.fori_loop(0, _CH, edge, acc)

        def ppi_step(c, acc):
            return chunk_sum(gene_hbm, gene_hbm, ps_hbm, pd_hbm,
                             wid * ppw + c * _CH, acc)

        acc_ppi = lax.fori_loop(0, ppi_chunks, ppi_step,
                                jnp.zeros((), jnp.float32))

        def dti_step(c, acc):
            return chunk_sum(drug_hbm, gene_hbm, ds_hbm, dd_hbm,
                             wid * dtw + c * _CH, acc)

        acc_dti = lax.fori_loop(0, dti_chunks, dti_step,
                                jnp.zeros((), jnp.float32))

        lane = lax.iota(jnp.int32, _L)
        ovec[0, :] = jnp.where(lane == 0, acc_ppi, 0.0)
        ovec[1, :] = jnp.where(lane == 0, acc_dti, 0.0)
        pltpu.sync_copy(ovec, out_hbm.at[wid])

    return edge_kernel


def _pad_idx(idx, total, fill):
    pad = total - idx.shape[0]
    if pad == 0:
        return idx.astype(jnp.int32)
    return jnp.concatenate(
        [idx.astype(jnp.int32),
         jnp.full((pad,), fill, dtype=jnp.int32)])


def kernel(gene_x, drug_x, predicted_dti, known_dti, ppi_edge_index,
           dti_src, dti_dst):
    dti_weight = 1.0
    topology_weight = 0.1

    n_gene, d = gene_x.shape
    n_drug = drug_x.shape[0]
    e_ppi = ppi_edge_index.shape[1]
    e_dti = predicted_dti.shape[0]

    info = plsc.get_sparse_core_info()
    nw = info.num_cores * info.num_subcores

    drug_rows = ((n_drug + _CH - 1) // _CH) * _CH + _CH
    drug_pad = jnp.concatenate(
        [drug_x, jnp.zeros((drug_rows - n_drug, d), drug_x.dtype)])
    gene_n = _normalize_rows(gene_x)
    drug_n = _normalize_rows(drug_pad)

    cols = 128
    n_flat = ((e_dti + cols * 8 - 1) // (cols * 8)) * (cols * 8)
    p2d = jnp.pad(predicted_dti, (0, n_flat - e_dti)).reshape(-1, cols)
    t2d = jnp.pad(known_dti, (0, n_flat - e_dti)).reshape(-1, cols)
    bce_total = _bce_sum(p2d, t2d, e_dti)

    per_block = nw * _CH

    def _even_chunks(n):
        c = (n + per_block - 1) // per_block
        return c + (c % 2)

    ppi_chunks = _even_chunks(e_ppi)
    dti_chunks = _even_chunks(e_dti)
    ppi_total = ppi_chunks * per_block
    dti_total = dti_chunks * per_block
    dti_pad = dti_total - e_dti

    ps = _pad_idx(ppi_edge_index[0], ppi_total, 0)
    pd = _pad_idx(ppi_edge_index[1], ppi_total, 0)
    ds = _pad_idx(dti_src, dti_total, n_drug)
    dd = _pad_idx(dti_dst, dti_total, 0)

    edge_kernel = _make_edge_kernel(nw, ppi_chunks, dti_chunks)
    parts = edge_kernel(gene_n, drug_n, ps, pd, ds, dd)

    ppi_sum = jnp.sum(parts[:, 0, :])
    dti_sum = jnp.sum(parts[:, 1, :]) - jnp.float32(dti_pad)

    topology_loss = ppi_sum / e_ppi + dti_sum / e_dti
    dti_loss = bce_total / e_dti
    return dti_weight * dti_loss + topology_weight * topology_loss


# speedup vs baseline: 1.5100x; 1.5100x over previous
import dataclasses
import functools

import jax
import jax.numpy as jnp
from jax import lax
from jax.experimental import pallas as pl
from jax.experimental.pallas import tpu as pltpu
from jax.experimental.pallas import tpu_sc as plsc

_L = 16
_CH = 128
_D = 128


def _norm_body(x_ref, o_ref):
    x = x_ref[...]
    ss = jnp.sum(x * x, axis=1, keepdims=True)
    n = jnp.sqrt(ss)
    o_ref[...] = x / jnp.maximum(n, 1e-12)


def _normalize_rows(x):
    return pl.pallas_call(
        _norm_body,
        out_shape=jax.ShapeDtypeStruct(x.shape, x.dtype),
    )(x)


def _bce_body(n_valid, p_ref, t_ref, o_ref):
    p = p_ref[...]
    t = t_ref[...]
    term = jnp.maximum(p, 0.0) - p * t + jnp.log1p(jnp.exp(-jnp.abs(p)))
    rows, cols = p.shape
    idx = (lax.broadcasted_iota(jnp.int32, (rows, cols), 0) * cols
           + lax.broadcasted_iota(jnp.int32, (rows, cols), 1))
    term = jnp.where(idx < n_valid, term, 0.0)
    o_ref[...] = jnp.sum(term, axis=0, keepdims=True)


def _bce_sum(p2d, t2d, n_valid):
    part = pl.pallas_call(
        functools.partial(_bce_body, n_valid),
        out_shape=jax.ShapeDtypeStruct((1, p2d.shape[1]), jnp.float32),
    )(p2d, t2d)
    return jnp.sum(part)


def _make_edge_kernel(nw, ppi_chunks, dti_chunks):
    ppw = ppi_chunks * _CH
    dtw = dti_chunks * _CH
    mesh = plsc.VectorSubcoreMesh(core_axis_name="c", subcore_axis_name="s")
    info = plsc.get_sparse_core_info()
    nc = info.num_cores

    cp = pltpu.CompilerParams()
    if "needs_layout_passes" in pltpu.CompilerParams.__dataclass_fields__:
        cp = dataclasses.replace(cp, needs_layout_passes=False)

    @functools.partial(
        pl.kernel,
        mesh=mesh,
        compiler_params=cp,
        out_type=jax.ShapeDtypeStruct((nw, 2, _L), jnp.float32),
        scratch_types=[
            pltpu.VMEM((_CH,), jnp.int32),
            pltpu.VMEM((_CH,), jnp.int32),
            pltpu.VMEM((_CH, _D), jnp.float32),
            pltpu.VMEM((_CH, _D), jnp.float32),
            pltpu.VMEM((2, _L), jnp.float32),
            pltpu.SemaphoreType.DMA,
            pltpu.SemaphoreType.DMA,
        ],
    )
    def edge_kernel(gene_hbm, drug_hbm, ps_hbm, pd_hbm, ds_hbm, dd_hbm,
                    out_hbm, sidx, didx, srows, drows, ovec, sem_a, sem_b):
        wid = lax.axis_index("s") * nc + lax.axis_index("c")

        def chunk_sum(src_tbl, dst_tbl, sidx_hbm, didx_hbm, base, acc):
            pltpu.sync_copy(sidx_hbm.at[pl.ds(base, _CH)], sidx)
            pltpu.sync_copy(didx_hbm.at[pl.ds(base, _CH)], didx)
            ca = pltpu.async_copy(src_tbl.at[sidx], srows, sem_a)
            cb = pltpu.async_copy(dst_tbl.at[didx], drows, sem_b)
            ca.wait()
            cb.wait()

            def edge(e, acc):
                prod = srows[e, pl.ds(0, _L)] * drows[e, pl.ds(0, _L)]
                for k in range(1, _D // _L):
                    prod = prod + (srows[e, pl.ds(k * _L, _L)]
                                   * drows[e, pl.ds(k * _L, _L)])
                dt = jnp.sum(prod)
                r = dt - 1.0
                return acc + r * r

            return lax.fori_loop(0, _CH, edge, acc)

        def ppi_step(c, acc):
            return chunk_sum(gene_hbm, gene_hbm, ps_hbm, pd_hbm,
                             wid * ppw + c * _CH, acc)

        acc_ppi = lax.fori_loop(0, ppi_chunks, ppi_step,
                                jnp.zeros((), jnp.float32))

        def dti_step(c, acc):
            return chunk_sum(drug_hbm, gene_hbm, ds_hbm, dd_hbm,
                             wid * dtw + c * _CH, acc)

        acc_dti = lax.fori_loop(0, dti_chunks, dti_step,
                                jnp.zeros((), jnp.float32))

        lane = lax.iota(jnp.int32, _L)
        ovec[0, :] = jnp.where(lane == 0, acc_ppi, 0.0)
        ovec[1, :] = jnp.where(lane == 0, acc_dti, 0.0)
        pltpu.sync_copy(ovec, out_hbm.at[wid])

    return edge_kernel


def _pad_idx(idx, total, fill):
    pad = total - idx.shape[0]
    if pad == 0:
        return idx.astype(jnp.int32)
    return jnp.concatenate(
        [idx.astype(jnp.int32),
         jnp.full((pad,), fill, dtype=jnp.int32)])


def kernel(gene_x, drug_x, predicted_dti, known_dti, ppi_edge_index,
           dti_src, dti_dst):
    dti_weight = 1.0
    topology_weight = 0.1

    n_gene, d = gene_x.shape
    n_drug = drug_x.shape[0]
    e_ppi = ppi_edge_index.shape[1]
    e_dti = predicted_dti.shape[0]

    info = plsc.get_sparse_core_info()
    nw = info.num_cores * info.num_subcores

    drug_rows = ((n_drug + _CH - 1) // _CH) * _CH + _CH
    drug_pad = jnp.concatenate(
        [drug_x, jnp.zeros((drug_rows - n_drug, d), drug_x.dtype)])
    gene_n = _normalize_rows(gene_x)
    drug_n = _normalize_rows(drug_pad)

    cols = 128
    n_flat = ((e_dti + cols * 8 - 1) // (cols * 8)) * (cols * 8)
    p2d = jnp.pad(predicted_dti, (0, n_flat - e_dti)).reshape(-1, cols)
    t2d = jnp.pad(known_dti, (0, n_flat - e_dti)).reshape(-1, cols)
    bce_total = _bce_sum(p2d, t2d, e_dti)

    per_block = nw * _CH

    def _even_chunks(n):
        return (n + per_block - 1) // per_block

    ppi_chunks = _even_chunks(e_ppi)
    dti_chunks = _even_chunks(e_dti)
    ppi_total = ppi_chunks * per_block
    dti_total = dti_chunks * per_block
    dti_pad = dti_total - e_dti

    ps = _pad_idx(ppi_edge_index[0], ppi_total, 0)
    pd = _pad_idx(ppi_edge_index[1], ppi_total, 0)
    ds = _pad_idx(dti_src, dti_total, n_drug)
    dd = _pad_idx(dti_dst, dti_total, 0)

    edge_kernel = _make_edge_kernel(nw, ppi_chunks, dti_chunks)
    parts = edge_kernel(gene_n, drug_n, ps, pd, ds, dd)

    ppi_sum = jnp.sum(parts[:, 0, :])
    dti_sum = jnp.sum(parts[:, 1, :]) - jnp.float32(dti_pad)

    topology_loss = ppi_sum / e_ppi + dti_sum / e_dti
    dti_loss = bce_total / e_dti
    return dti_weight * dti_loss + topology_weight * topology_loss
